# baseline (speedup 1.0000x reference)
import jax
import jax.numpy as jnp
from jax import lax
from jax.experimental import pallas as pl
from jax.experimental.pallas import tpu as pltpu
from jax.experimental.pallas import tpu_sc as plsc

N_ROWS = 320000
D = 128
N_SEG = 10000
N_SEG_PAD = 10240
NC = 2
NS = 16
NW = NC * NS
ROWS_PER_TILE = N_ROWS // NW
CHUNK = 80
N_CHUNKS = ROWS_PER_TILE // CHUNK
SEG_PER_TILE = N_SEG_PAD // NS


def _sc_body(y_hbm, idx_hbm, zero_hbm, zcnt_hbm, sums_hbm, cnts_hbm,
             idx_v, rows_v, cnt_v, acc_s,
             sem_i0, sem_i1, sem_r0, sem_r1, sem_s0, sem_s1):
    c = lax.axis_index("c")
    s = lax.axis_index("s")
    wid = c * NS + s
    row_base = wid * ROWS_PER_TILE
    idx_row_base = wid * N_CHUNKS
    sem_i = (sem_i0, sem_i1)
    sem_r = (sem_r0, sem_r1)
    sem_s = (sem_s0, sem_s1)

    def issue(j, b):
        pltpu.async_copy(idx_hbm.at[idx_row_base + j], idx_v.at[b], sem_i[b])
        pltpu.async_copy(y_hbm.at[pl.ds(row_base + j * CHUNK, CHUNK)],
                         rows_v.at[b], sem_r[b])

    def wait(j, b):
        pltpu.make_async_copy(
            idx_hbm.at[idx_row_base + j], idx_v.at[b], sem_i[b]).wait()
        pltpu.make_async_copy(
            y_hbm.at[pl.ds(row_base + j * CHUNK, CHUNK)],
            rows_v.at[b], sem_r[b]).wait()

    def consume(j, b):
        for k in range(CHUNK // 16):
            v = idx_v[b, pl.ds(k * 16, 16)]
            run, last = plsc.scan_count(v)
            plsc.addupdate_scatter(
                cnt_v, [v], run.astype(jnp.float32), mask=last)
        pltpu.async_copy(rows_v.at[b], acc_s.at[idx_v.at[b]], sem_s[b],
                         add=True)

    def wait_scatter(b):
        pltpu.make_async_copy(
            rows_v.at[b], acc_s.at[idx_v.at[b]], sem_s[b]).wait()

    pltpu.sync_copy(zcnt_hbm, cnt_v)
    zbase = s * SEG_PER_TILE
    pltpu.sync_copy(zero_hbm.at[pl.ds(zbase, SEG_PER_TILE)],
                    acc_s.at[pl.ds(zbase, SEG_PER_TILE)])
    plsc.subcore_barrier()

    issue(0, 0)
    issue(1, 1)
    def pair_body(p, _):
        j0 = 2 * p
        wait(j0, 0)
        consume(j0, 0)
        wait(j0 + 1, 1)
        consume(j0 + 1, 1)
        wait_scatter(0)
        @pl.when(j0 + 2 < N_CHUNKS)
        def _():
            issue(j0 + 2, 0)
        wait_scatter(1)
        @pl.when(j0 + 3 < N_CHUNKS)
        def _():
            issue(j0 + 3, 1)
        return 0
    lax.fori_loop(0, (N_CHUNKS - 1) // 2, pair_body, 0)
    wait(N_CHUNKS - 1, 0)
    consume(N_CHUNKS - 1, 0)
    wait_scatter(0)
    plsc.subcore_barrier()

    pltpu.sync_copy(acc_s.at[pl.ds(zbase, SEG_PER_TILE)],
                    sums_hbm.at[c, pl.ds(zbase, SEG_PER_TILE)])
    pltpu.sync_copy(cnt_v, cnts_hbm.at[wid])


_sc_scatter = pl.kernel(
    _sc_body,
    out_type=(
        jax.ShapeDtypeStruct((NC, N_SEG_PAD, D), jnp.float32),
        jax.ShapeDtypeStruct((NW, N_SEG_PAD), jnp.float32),
    ),
    mesh=plsc.VectorSubcoreMesh(
        core_axis_name="c", subcore_axis_name="s",
        num_cores=NC, num_subcores=NS),
    compiler_params=pltpu.CompilerParams(needs_layout_passes=False),
    scratch_types=[
        pltpu.VMEM((2, CHUNK), jnp.int32),
        pltpu.VMEM((2, CHUNK, D), jnp.float32),
        pltpu.VMEM((N_SEG_PAD,), jnp.float32),
        pltpu.VMEM_SHARED((N_SEG_PAD, D), jnp.float32),
        pltpu.SemaphoreType.DMA,
        pltpu.SemaphoreType.DMA,
        pltpu.SemaphoreType.DMA,
        pltpu.SemaphoreType.DMA,
        pltpu.SemaphoreType.DMA,
        pltpu.SemaphoreType.DMA,
    ],
)


def _combine_body(s_ref, c_ref, o_ref):
    tot = s_ref[0] + s_ref[1]
    cnt = jnp.sum(c_ref[...], axis=1, keepdims=True)
    o_ref[...] = tot * (1.0 / jnp.maximum(cnt, 1.0))


_COMBINE_B = 2048


def _combine(sums, cnts_t):
    grid = pl.cdiv(N_SEG, _COMBINE_B)
    return pl.pallas_call(
        _combine_body,
        grid=(grid,),
        in_specs=[
            pl.BlockSpec((NC, _COMBINE_B, D), lambda i: (0, i, 0)),
            pl.BlockSpec((_COMBINE_B, NW), lambda i: (i, 0)),
        ],
        out_specs=pl.BlockSpec((_COMBINE_B, D), lambda i: (i, 0)),
        out_shape=jax.ShapeDtypeStruct((N_SEG, D), jnp.float32),
    )(sums, cnts_t)


@jax.jit
def kernel(y, idx):
    idx32 = idx.astype(jnp.int32).reshape(N_ROWS // CHUNK, CHUNK)
    zero = jnp.zeros((N_SEG_PAD, D), jnp.float32)
    zcnt = jnp.zeros((N_SEG_PAD,), jnp.float32)
    sums, cnts = _sc_scatter(y, idx32, zero, zcnt)
    return _combine(sums, cnts.T)


# speedup vs baseline: 1.1748x; 1.1748x over previous
import jax
import jax.numpy as jnp
from jax import lax
from jax.experimental import pallas as pl
from jax.experimental.pallas import tpu as pltpu
from jax.experimental.pallas import tpu_sc as plsc

N_ROWS = 320000
D = 128
N_SEG = 10000
N_SEG_PAD = 10240
NC = 2
NS = 16
NW = NC * NS
ROWS_PER_TILE = N_ROWS // NW
CHUNK = 80
N_CHUNKS = ROWS_PER_TILE // CHUNK
SEG_PER_TILE = N_SEG_PAD // NS


def _sums_body(y_hbm, idx_hbm, zero_hbm, sums_hbm,
               idx_all, rows_v, acc_s, sem_r0, sem_r1):
    c = lax.axis_index("c")
    s = lax.axis_index("s")
    wid = c * NS + s
    row_base = wid * ROWS_PER_TILE
    sem_r = (sem_r0, sem_r1)

    def issue(j, b):
        pltpu.async_copy(y_hbm.at[pl.ds(row_base + j * CHUNK, CHUNK)],
                         rows_v.at[b], sem_r[b])

    def wait(j, b):
        pltpu.make_async_copy(
            y_hbm.at[pl.ds(row_base + j * CHUNK, CHUNK)],
            rows_v.at[b], sem_r[b]).wait()

    pltpu.async_copy(idx_hbm.at[wid], idx_all, sem_r0)
    issue(0, 1)
    zbase = s * SEG_PER_TILE
    pltpu.sync_copy(zero_hbm.at[pl.ds(zbase, SEG_PER_TILE)],
                    acc_s.at[pl.ds(zbase, SEG_PER_TILE)])
    pltpu.make_async_copy(idx_hbm.at[wid], idx_all, sem_r0).wait()
    plsc.subcore_barrier()

    issue(1, 0)
    def pair_body(p, _):
        j0 = 2 * p
        wait(j0, 1)
        pltpu.sync_copy(rows_v.at[1], acc_s.at[idx_all.at[j0]], add=True)
        issue(j0 + 2, 1)
        wait(j0 + 1, 0)
        pltpu.sync_copy(rows_v.at[0], acc_s.at[idx_all.at[j0 + 1]], add=True)
        @pl.when(j0 + 3 < N_CHUNKS)
        def _():
            issue(j0 + 3, 0)
        return 0
    lax.fori_loop(0, (N_CHUNKS - 1) // 2, pair_body, 0)
    wait(N_CHUNKS - 1, 1)
    pltpu.sync_copy(rows_v.at[1], acc_s.at[idx_all.at[N_CHUNKS - 1]],
                    add=True)
    plsc.subcore_barrier()

    pltpu.sync_copy(acc_s.at[pl.ds(zbase, SEG_PER_TILE)],
                    sums_hbm.at[c, pl.ds(zbase, SEG_PER_TILE)])


_sc_sums = pl.kernel(
    _sums_body,
    out_type=jax.ShapeDtypeStruct((NC, N_SEG_PAD, D), jnp.float32),
    mesh=plsc.VectorSubcoreMesh(
        core_axis_name="c", subcore_axis_name="s",
        num_cores=NC, num_subcores=NS),
    compiler_params=pltpu.CompilerParams(needs_layout_passes=False),
    scratch_types=[
        pltpu.VMEM((N_CHUNKS, CHUNK), jnp.int32),
        pltpu.VMEM((2, CHUNK, D), jnp.float32),
        pltpu.VMEM_SHARED((N_SEG_PAD, D), jnp.float32),
        pltpu.SemaphoreType.DMA,
        pltpu.SemaphoreType.DMA,
    ],
)


def _counts_body(idx_hbm, zcnt_hbm, cnts_hbm, idx_all, cnt_v):
    c = lax.axis_index("c")
    s = lax.axis_index("s")
    wid = c * NS + s
    pltpu.sync_copy(zcnt_hbm, cnt_v)
    pltpu.sync_copy(idx_hbm.at[wid], idx_all)

    def vec_body(i, _):
        v = idx_all[i // (CHUNK // 16), pl.ds((i % (CHUNK // 16)) * 16, 16)]
        run, last = plsc.scan_count(v)
        plsc.addupdate_scatter(
            cnt_v, [v], run.astype(jnp.float32), mask=last)
        return 0
    lax.fori_loop(0, ROWS_PER_TILE // 16, vec_body, 0)
    pltpu.sync_copy(cnt_v, cnts_hbm.at[wid])


_sc_counts = pl.kernel(
    _counts_body,
    out_type=jax.ShapeDtypeStruct((NW, N_SEG_PAD), jnp.float32),
    mesh=plsc.VectorSubcoreMesh(
        core_axis_name="c", subcore_axis_name="s",
        num_cores=NC, num_subcores=NS),
    compiler_params=pltpu.CompilerParams(needs_layout_passes=False),
    scratch_types=[
        pltpu.VMEM((N_CHUNKS, CHUNK), jnp.int32),
        pltpu.VMEM((N_SEG_PAD,), jnp.float32),
    ],
)


def _combine_body(s_ref, c_ref, o_ref):
    tot = s_ref[0] + s_ref[1]
    cnt = jnp.sum(c_ref[...], axis=1, keepdims=True)
    o_ref[...] = tot * (1.0 / jnp.maximum(cnt, 1.0))


_COMBINE_B = 2048


def _combine(sums, cnts_t):
    grid = pl.cdiv(N_SEG, _COMBINE_B)
    return pl.pallas_call(
        _combine_body,
        grid=(grid,),
        in_specs=[
            pl.BlockSpec((NC, _COMBINE_B, D), lambda i: (0, i, 0)),
            pl.BlockSpec((_COMBINE_B, NW), lambda i: (i, 0)),
        ],
        out_specs=pl.BlockSpec((_COMBINE_B, D), lambda i: (i, 0)),
        out_shape=jax.ShapeDtypeStruct((N_SEG, D), jnp.float32),
    )(sums, cnts_t)


@jax.jit
def kernel(y, idx):
    idx3 = idx.astype(jnp.int32).reshape(NW, N_CHUNKS, CHUNK)
    zero = jnp.zeros((N_SEG_PAD, D), jnp.float32)
    zcnt = jnp.zeros((N_SEG_PAD,), jnp.float32)
    sums = _sc_sums(y, idx3, zero)
    cnts = _sc_counts(idx3, zcnt)
    return _combine(sums, cnts.T)
